# baseline (speedup 1.0000x reference)
import functools

import jax
import jax.numpy as jnp
from jax import lax
from jax.experimental import pallas as pl
from jax.experimental.pallas import tpu as pltpu
from jax.experimental.pallas import tpu_sc as plsc

NC = 2
NS = 16
NW = NC * NS
L = 16
HB = 8


def _scatter_kernel(M, D, B):
    blocks = M // HB
    assert blocks * HB == M
    BLO = blocks // NW
    EXTRA = blocks % NW
    RMAX = (BLO + 1) * HB
    WCAP = ((RMAX + L - 1) // L) * L
    LCAP = WCAP + 2 * L
    CC = 120
    NFULL = (BLO * HB) // CC
    assert NFULL * CC == BLO * HB
    NSCAN = B // L
    SCANK = -(-NSCAN // NFULL)
    UCAP = LCAP // L

    mesh = plsc.VectorSubcoreMesh(core_axis_name="c", subcore_axis_name="s")

    @functools.partial(
        pl.kernel,
        out_type=jax.ShapeDtypeStruct((M, D), jnp.float32),
        mesh=mesh,
        compiler_params=pltpu.CompilerParams(needs_layout_passes=False),
        scratch_types=[
            pltpu.VMEM((B,), jnp.int32),
            pltpu.VMEM((WCAP,), jnp.int32),
            pltpu.VMEM((LCAP,), jnp.int32),
            pltpu.VMEM((LCAP,), jnp.int32),
            pltpu.VMEM((4, CC, D), jnp.float32),
            pltpu.VMEM((HB, D), jnp.float32),
            pltpu.VMEM((4, L, D), jnp.float32),
            pltpu.SemaphoreType.DMA,
            pltpu.SemaphoreType.DMA,
            pltpu.SemaphoreType.DMA,
            pltpu.SemaphoreType.DMA,
            pltpu.SemaphoreType.DMA,
        ],
    )
    def k(sav, idx_h, qp, out, idx_v, winner, rowl, posl, cbuf, tbuf, gb,
          sem_idx, sem_r, sem_w, sem_g, sem_s):
        wid = lax.axis_index("s") * NC + lax.axis_index("c")
        base = (wid * BLO + jnp.minimum(wid, EXTRA)) * HB
        rows_w = (BLO + jnp.where(wid < EXTRA, 1, 0)) * HB
        iota = lax.iota(jnp.int32, L)

        def rd(t):
            return pltpu.make_async_copy(
                sav.at[pl.ds(base + t * CC, CC)], cbuf.at[t % 4], sem_r)

        def wr(t):
            return pltpu.make_async_copy(
                cbuf.at[t % 4], out.at[pl.ds(base + t * CC, CC)], sem_w)

        cp_idx = pltpu.async_copy(idx_h, idx_v, sem_idx)
        rd(0).start()
        rd(1).start()
        rd(2).start()

        minus1 = jnp.full((L,), -1, jnp.int32)

        def initb(i, c):
            winner[pl.ds(i * L, L)] = minus1
            return c

        lax.fori_loop(0, WCAP // L, initb, 0)
        cp_idx.wait()

        def scan_chunk(i):
            p0 = i * L
            pv = iota + p0
            r = idx_v[pl.ds(p0, L)] - base
            m = (r >= 0) & (r < rows_w)

            @pl.when(jnp.any(m))
            def _():
                rc = jnp.where(m, r, 0)
                plsc.store_scatter(winner, [rc], pv, mask=m)
                w0 = plsc.load_gather(winner, [rc], mask=m)

                def cond(w):
                    return jnp.any(m & (w < pv))

                def retry(w):
                    plsc.store_scatter(winner, [rc], pv, mask=m & (w < pv))
                    return plsc.load_gather(winner, [rc], mask=m)

                lax.while_loop(cond, retry, w0)

        def copyb(t, c):
            def scanb(j, c2):
                sidx = t * SCANK + 2 * j
                for u in range(2):
                    @pl.when(sidx + u < NSCAN)
                    def _():
                        scan_chunk(sidx + u)

                return c2

            lax.fori_loop(0, SCANK // 2, scanb, 0)

            rd(t).wait()
            wr(t).start()

            @pl.when(t >= 1)
            def _():
                wr(t - 1).wait()

            @pl.when(t + 3 < NFULL)
            def _():
                rd(t + 3).start()

            return c

        lax.fori_loop(0, NFULL, copyb, 0)
        wr(0).wait()

        @pl.when(rows_w > NFULL * CC)
        def _():
            tb = base + NFULL * CC
            pltpu.async_copy(sav.at[pl.ds(tb, HB)], tbuf, sem_r).wait()
            pltpu.async_copy(tbuf, out.at[pl.ds(tb, HB)], sem_w).wait()

        def compb(j, carry):
            off, lastr = carry
            rv = iota + j * L
            wv = winner[pl.ds(j * L, L)]
            m = wv >= 0
            plsc.store_compressed(rowl.at[pl.ds(off, L)], rv + base, mask=m)
            plsc.store_compressed(posl.at[pl.ds(off, L)], wv, mask=m)
            cnt = jnp.sum(m.astype(jnp.int32))
            lastr = jnp.maximum(lastr, jnp.max(jnp.where(m, rv, -1)))
            return off + cnt, lastr

        n_upd, lastr = lax.fori_loop(0, WCAP // L, compb, (0, -1))

        lastr_vec = jnp.broadcast_to(jnp.maximum(lastr, 0), (L,))
        lastp_vec = plsc.load_gather(winner, [lastr_vec])
        rowl[pl.ds(n_upd, L)] = lastr_vec + base
        posl[pl.ds(n_upd, L)] = lastp_vec

        nch_u = (n_upd + L - 1) // L

        def g(t):
            pv = posl[pl.ds(t * L, L)]
            return pltpu.make_async_copy(qp.at[pv], gb.at[t % 4], sem_g)

        def sca(t):
            rv = rowl[pl.ds(t * L, L)]
            return pltpu.make_async_copy(gb.at[t % 4], out.at[rv], sem_s)

        @pl.when(nch_u > 0)
        def _():
            g(0).start()

        @pl.when(nch_u > 1)
        def _():
            g(1).start()

        def ubody(t, c):
            @pl.when(t < nch_u)
            def _():
                @pl.when(t >= 2)
                def _():
                    sca(t - 2).wait()

                @pl.when(t + 2 < nch_u)
                def _():
                    g(t + 2).start()

                g(t).wait()
                sca(t).start()

            return c

        lax.fori_loop(0, UCAP, ubody, 0)

        @pl.when(nch_u >= 1)
        def _():
            sca(0).wait()

        @pl.when(nch_u >= 2)
        def _():
            sca(0).wait()

    return k


def kernel(state_action_values, action, q_prime):
    M, D = state_action_values.shape
    B = q_prime.shape[0]
    idx = action.reshape(B)
    k = _scatter_kernel(M, D, B)
    return k(state_action_values, idx, q_prime)


# speedup vs baseline: 1.2087x; 1.2087x over previous
import functools

import jax
import jax.numpy as jnp
from jax import lax
from jax.experimental import pallas as pl
from jax.experimental.pallas import tpu as pltpu
from jax.experimental.pallas import tpu_sc as plsc

NC = 2
NS = 16
NW = NC * NS
L = 16
HB = 8


def _scatter_kernel(M, D, B):
    blocks = M // HB
    assert blocks * HB == M
    BLO = blocks // NW
    EXTRA = blocks % NW
    RMAX = (BLO + 1) * HB
    WCAP = ((RMAX + L - 1) // L) * L
    LCAP = WCAP + 2 * L
    CC = 120
    NFULL = (BLO * HB) // CC
    assert NFULL * CC == BLO * HB
    NSCAN = B // L
    SCANK = -(-NSCAN // NFULL)
    UCAP = LCAP // L

    mesh = plsc.VectorSubcoreMesh(core_axis_name="c", subcore_axis_name="s")

    @functools.partial(
        pl.kernel,
        out_type=jax.ShapeDtypeStruct((M, D), jnp.float32),
        mesh=mesh,
        compiler_params=pltpu.CompilerParams(needs_layout_passes=False),
        scratch_types=[
            pltpu.VMEM((B,), jnp.int32),
            pltpu.VMEM((WCAP,), jnp.int32),
            pltpu.VMEM((LCAP,), jnp.int32),
            pltpu.VMEM((LCAP,), jnp.int32),
            pltpu.VMEM((4, CC, D), jnp.float32),
            pltpu.VMEM((HB, D), jnp.float32),
            pltpu.VMEM((4, L, D), jnp.float32),
            pltpu.SemaphoreType.DMA,
            pltpu.SemaphoreType.DMA,
            pltpu.SemaphoreType.DMA,
            pltpu.SemaphoreType.DMA,
            pltpu.SemaphoreType.DMA,
        ],
    )
    def k(sav, idx_h, qp, out, idx_v, winner, rowl, posl, cbuf, tbuf, gb,
          sem_idx, sem_r, sem_w, sem_g, sem_s):
        wid = lax.axis_index("s") * NC + lax.axis_index("c")
        base = (wid * BLO + jnp.minimum(wid, EXTRA)) * HB
        rows_w = (BLO + jnp.where(wid < EXTRA, 1, 0)) * HB
        iota = lax.iota(jnp.int32, L)

        def rd(t):
            return pltpu.make_async_copy(
                sav.at[pl.ds(base + t * CC, CC)], cbuf.at[t % 4], sem_r)

        def wr(t):
            return pltpu.make_async_copy(
                cbuf.at[t % 4], out.at[pl.ds(base + t * CC, CC)], sem_w)

        cp_idx = pltpu.async_copy(idx_h, idx_v, sem_idx)
        rd(0).start()
        rd(1).start()
        rd(2).start()

        minus1 = jnp.full((L,), -1, jnp.int32)

        def initb(i, c):
            winner[pl.ds(i * L, L)] = minus1
            return c

        lax.fori_loop(0, WCAP // L, initb, 0)
        cp_idx.wait()

        def scan_chunk(i):
            p0 = i * L
            pv = iota + p0
            r = idx_v[pl.ds(p0, L)] - base
            m = (r >= 0) & (r < rows_w)
            rc = jnp.where(m, r, 0)
            plsc.store_scatter(winner, [rc], pv, mask=m)
            w0 = plsc.load_gather(winner, [rc], mask=m)

            def cond(w):
                return jnp.any(m & (w < pv))

            def retry(w):
                plsc.store_scatter(winner, [rc], pv, mask=m & (w < pv))
                return plsc.load_gather(winner, [rc], mask=m)

            lax.while_loop(cond, retry, w0)

        def copyb(t, c):
            def scanb(j, c2):
                sidx = t * SCANK + 2 * j
                for u in range(2):
                    @pl.when(sidx + u < NSCAN)
                    def _():
                        scan_chunk(sidx + u)

                return c2

            lax.fori_loop(0, SCANK // 2, scanb, 0)

            rd(t).wait()
            wr(t).start()

            @pl.when(t >= 1)
            def _():
                wr(t - 1).wait()

            @pl.when(t + 3 < NFULL)
            def _():
                rd(t + 3).start()

            return c

        lax.fori_loop(0, NFULL, copyb, 0)
        wr(0).wait()

        @pl.when(rows_w > NFULL * CC)
        def _():
            tb = base + NFULL * CC
            pltpu.async_copy(sav.at[pl.ds(tb, HB)], tbuf, sem_r).wait()
            pltpu.async_copy(tbuf, out.at[pl.ds(tb, HB)], sem_w).wait()

        def compb(j, carry):
            off, lastr = carry
            rv = iota + j * L
            wv = winner[pl.ds(j * L, L)]
            m = wv >= 0
            plsc.store_compressed(rowl.at[pl.ds(off, L)], rv + base, mask=m)
            plsc.store_compressed(posl.at[pl.ds(off, L)], wv, mask=m)
            cnt = jnp.sum(m.astype(jnp.int32))
            lastr = jnp.maximum(lastr, jnp.max(jnp.where(m, rv, -1)))
            return off + cnt, lastr

        n_upd, lastr = lax.fori_loop(0, WCAP // L, compb, (0, -1))

        lastr_vec = jnp.broadcast_to(jnp.maximum(lastr, 0), (L,))
        lastp_vec = plsc.load_gather(winner, [lastr_vec])
        rowl[pl.ds(n_upd, L)] = lastr_vec + base
        posl[pl.ds(n_upd, L)] = lastp_vec

        nch_u = (n_upd + L - 1) // L

        def g(t):
            pv = posl[pl.ds(t * L, L)]
            return pltpu.make_async_copy(qp.at[pv], gb.at[t % 4], sem_g)

        def sca(t):
            rv = rowl[pl.ds(t * L, L)]
            return pltpu.make_async_copy(gb.at[t % 4], out.at[rv], sem_s)

        @pl.when(nch_u > 0)
        def _():
            g(0).start()

        @pl.when(nch_u > 1)
        def _():
            g(1).start()

        def ubody(t, c):
            @pl.when(t < nch_u)
            def _():
                @pl.when(t >= 2)
                def _():
                    sca(t - 2).wait()

                @pl.when(t + 2 < nch_u)
                def _():
                    g(t + 2).start()

                g(t).wait()
                sca(t).start()

            return c

        lax.fori_loop(0, UCAP, ubody, 0)

        @pl.when(nch_u >= 1)
        def _():
            sca(0).wait()

        @pl.when(nch_u >= 2)
        def _():
            sca(0).wait()

    return k


def kernel(state_action_values, action, q_prime):
    M, D = state_action_values.shape
    B = q_prime.shape[0]
    idx = action.reshape(B)
    k = _scatter_kernel(M, D, B)
    return k(state_action_values, idx, q_prime)
